# Initial kernel scaffold; baseline (speedup 1.0000x reference)
import jax
import jax.numpy as jnp
from jax.experimental import pallas as pl


def kernel(r_ids, W):
    raise NotImplementedError("write your pallas kernel here")


# speedup vs baseline: 101.4177x; 101.4177x over previous
import functools

import jax
import jax.numpy as jnp
from jax import lax
from jax.experimental import pallas as pl
from jax.experimental.pallas import tpu as pltpu
from jax.experimental.pallas import tpu_sc as plsc

_LANES = 16


def _sc_workers():
    try:
        info = plsc.get_sparse_core_info()
        return info.num_cores, info.num_subcores
    except Exception:
        return 2, 16


def kernel(r_ids, W):
    B, F = r_ids.shape
    V = W.shape[0]
    N = B * F

    idx = r_ids.reshape(N).astype(jnp.int32)
    table = W.reshape(V)

    NC, NS = _sc_workers()
    NW = NC * NS
    per_w = N // NW
    assert per_w * NW == N
    chunk = 12800
    n_chunks = per_w // chunk
    assert n_chunks * chunk == per_w

    mesh = plsc.VectorSubcoreMesh(
        core_axis_name="c", subcore_axis_name="s",
        num_cores=NC, num_subcores=NS,
    )

    @functools.partial(
        pl.kernel,
        out_type=jax.ShapeDtypeStruct((N,), jnp.float32),
        mesh=mesh,
        compiler_params=pltpu.CompilerParams(needs_layout_passes=False),
        scratch_types=[
            pltpu.VMEM((V,), jnp.float32),
            pltpu.VMEM((chunk,), jnp.int32),
            pltpu.VMEM((chunk,), jnp.float32),
        ],
    )
    def sc_gather(idx_hbm, tab_hbm, out_hbm, tab_v, idx_v, out_v):
        wid = lax.axis_index("s") * NC + lax.axis_index("c")
        base = wid * per_w
        pltpu.sync_copy(tab_hbm, tab_v)

        def chunk_body(k, carry):
            off = base + k * chunk
            pltpu.sync_copy(idx_hbm.at[pl.ds(off, chunk)], idx_v)

            def body(i, c):
                iv = idx_v[pl.ds(i * _LANES, _LANES)]
                vals = plsc.load_gather(tab_v, [iv])
                out_v[pl.ds(i * _LANES, _LANES)] = 2.0 / (1.0 + jnp.exp(-vals))
                return c

            lax.fori_loop(0, chunk // _LANES, body, 0)
            pltpu.sync_copy(out_v, out_hbm.at[pl.ds(off, chunk)])
            return carry

        lax.fori_loop(0, n_chunks, chunk_body, 0)

    out = sc_gather(idx, table)
    return out.reshape(B, F)


# speedup vs baseline: 174.8081x; 1.7236x over previous
import functools

import jax
import jax.numpy as jnp
from jax import lax
from jax.experimental import pallas as pl
from jax.experimental.pallas import tpu as pltpu
from jax.experimental.pallas import tpu_sc as plsc

_LANES = 16
_CHUNK = 6400


def _sc_workers():
    try:
        info = plsc.get_sparse_core_info()
        return info.num_cores, info.num_subcores
    except Exception:
        return 2, 16


def _tc_table_transform(w2d):
    def body(w_ref, t_ref):
        t_ref[...] = 2.0 / (1.0 + jnp.exp(-w_ref[...]))

    return pl.pallas_call(
        body,
        out_shape=jax.ShapeDtypeStruct(w2d.shape, jnp.float32),
    )(w2d)


def kernel(r_ids, W):
    B, F = r_ids.shape
    V = W.shape[0]
    N = B * F
    Vp = -(-V // 1024) * 1024

    w_pad = jnp.pad(W.reshape(V), (0, Vp - V))
    table = _tc_table_transform(w_pad.reshape(Vp // 128, 128)).reshape(Vp)
    idx = r_ids.reshape(N).astype(jnp.int32)

    NC, NS = _sc_workers()
    NW = NC * NS
    per_w = N // NW
    assert per_w * NW == N
    n_chunks = per_w // _CHUNK
    assert n_chunks * _CHUNK == per_w and n_chunks >= 2

    mesh = plsc.VectorSubcoreMesh(
        core_axis_name="c", subcore_axis_name="s",
        num_cores=NC, num_subcores=NS,
    )

    @functools.partial(
        pl.kernel,
        out_type=jax.ShapeDtypeStruct((N,), jnp.float32),
        mesh=mesh,
        compiler_params=pltpu.CompilerParams(needs_layout_passes=False),
        scratch_types=[
            pltpu.VMEM((Vp,), jnp.float32),
            pltpu.VMEM((_CHUNK,), jnp.int32),
            pltpu.VMEM((_CHUNK,), jnp.int32),
            pltpu.VMEM((_CHUNK,), jnp.float32),
            pltpu.VMEM((_CHUNK,), jnp.float32),
            pltpu.SemaphoreType.DMA,
            pltpu.SemaphoreType.DMA,
            pltpu.SemaphoreType.DMA,
            pltpu.SemaphoreType.DMA,
            pltpu.SemaphoreType.DMA,
        ],
    )
    def sc_gather(idx_hbm, tab_hbm, out_hbm, tab_v, idx_a, idx_b, out_a,
                  out_b, sem_t, sem_ia, sem_ib, sem_oa, sem_ob):
        wid = lax.axis_index("s") * NC + lax.axis_index("c")
        base = wid * per_w

        idx_bufs = (idx_a, idx_b)
        out_bufs = (out_a, out_b)
        idx_sems = (sem_ia, sem_ib)
        out_sems = (sem_oa, sem_ob)

        tab_cp = pltpu.async_copy(tab_hbm, tab_v, sem_t)
        idx_cps = [None] * n_chunks
        out_cps = [None] * n_chunks
        idx_cps[0] = pltpu.async_copy(
            idx_hbm.at[pl.ds(base, _CHUNK)], idx_bufs[0], idx_sems[0])
        tab_cp.wait()

        for k in range(n_chunks):
            p = k % 2
            idx_cps[k].wait()
            if k + 1 < n_chunks:
                idx_cps[k + 1] = pltpu.async_copy(
                    idx_hbm.at[pl.ds(base + (k + 1) * _CHUNK, _CHUNK)],
                    idx_bufs[(k + 1) % 2], idx_sems[(k + 1) % 2])
            if k >= 2:
                out_cps[k - 2].wait()
            ib, ob = idx_bufs[p], out_bufs[p]

            @plsc.parallel_loop(0, _CHUNK // _LANES, unroll=8)
            def gather_body(i, ib=ib, ob=ob):
                iv = ib[pl.ds(i * _LANES, _LANES)]
                ob[pl.ds(i * _LANES, _LANES)] = plsc.load_gather(tab_v, [iv])

            out_cps[k] = pltpu.async_copy(
                ob, out_hbm.at[pl.ds(base + k * _CHUNK, _CHUNK)], out_sems[p])

        out_cps[n_chunks - 2].wait()
        out_cps[n_chunks - 1].wait()

    out = sc_gather(idx, table)
    return out.reshape(B, F)


# speedup vs baseline: 231.0723x; 1.3219x over previous
import functools

import jax
import jax.numpy as jnp
from jax import lax
from jax.experimental import pallas as pl
from jax.experimental.pallas import tpu as pltpu
from jax.experimental.pallas import tpu_sc as plsc

_LANES = 16
_ROWS = 32


def _sc_workers():
    try:
        info = plsc.get_sparse_core_info()
        return info.num_cores, info.num_subcores
    except Exception:
        return 2, 16


def _tc_table_transform(w2d):
    def body(w_ref, t_ref):
        t_ref[...] = 2.0 / (1.0 + jnp.exp(-w_ref[...]))

    return pl.pallas_call(
        body,
        out_shape=jax.ShapeDtypeStruct(w2d.shape, jnp.float32),
    )(w2d)


def kernel(r_ids, W):
    B, F = r_ids.shape
    V = W.shape[0]
    Vp = -(-V // 1024) * 1024
    col_offs = list(range(0, F - _LANES + 1, _LANES))
    if col_offs[-1] + _LANES < F:
        col_offs.append(F - _LANES)

    w_pad = jnp.pad(W.reshape(V), (0, Vp - V))
    table = _tc_table_transform(w_pad.reshape(Vp // 128, 128)).reshape(Vp)
    idx = r_ids.astype(jnp.int32)

    NC, NS = _sc_workers()
    NW = NC * NS
    rows_per_w = B // NW
    assert rows_per_w * NW == B
    n_chunks = rows_per_w // _ROWS
    assert n_chunks * _ROWS == rows_per_w and n_chunks >= 2

    mesh = plsc.VectorSubcoreMesh(
        core_axis_name="c", subcore_axis_name="s",
        num_cores=NC, num_subcores=NS,
    )

    @functools.partial(
        pl.kernel,
        out_type=jax.ShapeDtypeStruct((B, F), jnp.float32),
        mesh=mesh,
        compiler_params=pltpu.CompilerParams(needs_layout_passes=False),
        scratch_types=[
            pltpu.VMEM((Vp,), jnp.float32),
            pltpu.VMEM((_ROWS, F), jnp.int32),
            pltpu.VMEM((_ROWS, F), jnp.int32),
            pltpu.VMEM((_ROWS, F), jnp.float32),
            pltpu.VMEM((_ROWS, F), jnp.float32),
            pltpu.SemaphoreType.DMA,
            pltpu.SemaphoreType.DMA,
            pltpu.SemaphoreType.DMA,
            pltpu.SemaphoreType.DMA,
            pltpu.SemaphoreType.DMA,
        ],
    )
    def sc_gather(idx_hbm, tab_hbm, out_hbm, tab_v, idx_a, idx_b, out_a,
                  out_b, sem_t, sem_ia, sem_ib, sem_oa, sem_ob):
        wid = lax.axis_index("s") * NC + lax.axis_index("c")
        row0 = wid * rows_per_w

        idx_bufs = (idx_a, idx_b)
        out_bufs = (out_a, out_b)
        idx_sems = (sem_ia, sem_ib)
        out_sems = (sem_oa, sem_ob)

        tab_cp = pltpu.async_copy(tab_hbm, tab_v, sem_t)
        idx_cps = [None] * n_chunks
        out_cps = [None] * n_chunks
        idx_cps[0] = pltpu.async_copy(
            idx_hbm.at[pl.ds(row0, _ROWS), :], idx_bufs[0], idx_sems[0])
        tab_cp.wait()

        for k in range(n_chunks):
            p = k % 2
            idx_cps[k].wait()
            if k + 1 < n_chunks:
                idx_cps[k + 1] = pltpu.async_copy(
                    idx_hbm.at[pl.ds(row0 + (k + 1) * _ROWS, _ROWS), :],
                    idx_bufs[(k + 1) % 2], idx_sems[(k + 1) % 2])
            if k >= 2:
                out_cps[k - 2].wait()
            ib, ob = idx_bufs[p], out_bufs[p]

            @plsc.parallel_loop(0, _ROWS, unroll=2)
            def gather_body(r, ib=ib, ob=ob):
                for c in col_offs:
                    iv = ib[r, pl.ds(c, _LANES)]
                    ob[r, pl.ds(c, _LANES)] = plsc.load_gather(tab_v, [iv])

            out_cps[k] = pltpu.async_copy(
                ob, out_hbm.at[pl.ds(row0 + k * _ROWS, _ROWS), :], out_sems[p])

        out_cps[n_chunks - 2].wait()
        out_cps[n_chunks - 1].wait()

    out = sc_gather(idx, table)
    return out


# speedup vs baseline: 250.5219x; 1.0842x over previous
import functools

import jax
import jax.numpy as jnp
from jax import lax
from jax.experimental import pallas as pl
from jax.experimental.pallas import tpu as pltpu
from jax.experimental.pallas import tpu_sc as plsc

_LANES = 16
_ROWS = 32


def _sc_workers():
    try:
        info = plsc.get_sparse_core_info()
        return info.num_cores, info.num_subcores
    except Exception:
        return 2, 16


def _tc_table_transform(w2d):
    def body(w_ref, t_ref):
        t_ref[...] = 2.0 / (1.0 + jnp.exp(-w_ref[...]))

    return pl.pallas_call(
        body,
        out_shape=jax.ShapeDtypeStruct(w2d.shape, jnp.float32),
    )(w2d)


def kernel(r_ids, W):
    B, F = r_ids.shape
    V = W.shape[0]
    Vp = -(-V // 1024) * 1024
    col_offs = list(range(0, F - _LANES + 1, _LANES))
    if col_offs[-1] + _LANES < F:
        col_offs.append(F - _LANES)

    w_pad = jnp.pad(W.reshape(V), (0, Vp - V))
    table = _tc_table_transform(w_pad.reshape(Vp // 128, 128)).reshape(Vp)
    idx = r_ids.astype(jnp.int32)

    NC, NS = _sc_workers()
    NW = NC * NS
    rows_per_w = B // NW
    assert rows_per_w * NW == B
    n_chunks = rows_per_w // _ROWS
    assert n_chunks * _ROWS == rows_per_w and n_chunks >= 2

    mesh = plsc.VectorSubcoreMesh(
        core_axis_name="c", subcore_axis_name="s",
        num_cores=NC, num_subcores=NS,
    )

    @functools.partial(
        pl.kernel,
        out_type=jax.ShapeDtypeStruct((B, F), jnp.float32),
        mesh=mesh,
        compiler_params=pltpu.CompilerParams(needs_layout_passes=False),
        scratch_types=[
            pltpu.VMEM((Vp,), jnp.float32),
            pltpu.VMEM((_ROWS, F), jnp.int32),
            pltpu.VMEM((_ROWS, F), jnp.int32),
            pltpu.VMEM((_ROWS, F), jnp.int32),
            pltpu.VMEM((_ROWS, F), jnp.float32),
            pltpu.VMEM((_ROWS, F), jnp.float32),
            pltpu.VMEM((_ROWS, F), jnp.float32),
            pltpu.SemaphoreType.DMA,
            pltpu.SemaphoreType.DMA,
            pltpu.SemaphoreType.DMA,
            pltpu.SemaphoreType.DMA,
            pltpu.SemaphoreType.DMA,
            pltpu.SemaphoreType.DMA,
            pltpu.SemaphoreType.DMA,
        ],
    )
    def sc_gather(idx_hbm, tab_hbm, out_hbm, tab_v, idx_a, idx_b, idx_c,
                  out_a, out_b, out_c, sem_t, sem_ia, sem_ib, sem_ic,
                  sem_oa, sem_ob, sem_oc):
        wid = lax.axis_index("s") * NC + lax.axis_index("c")
        row0 = wid * rows_per_w

        nbuf = 3
        idx_bufs = (idx_a, idx_b, idx_c)
        out_bufs = (out_a, out_b, out_c)
        idx_sems = (sem_ia, sem_ib, sem_ic)
        out_sems = (sem_oa, sem_ob, sem_oc)

        tab_cp = pltpu.async_copy(tab_hbm, tab_v, sem_t)
        idx_cps = [None] * n_chunks
        out_cps = [None] * n_chunks
        for k in range(nbuf):
            idx_cps[k] = pltpu.async_copy(
                idx_hbm.at[pl.ds(row0 + k * _ROWS, _ROWS), :],
                idx_bufs[k], idx_sems[k])
        tab_cp.wait()

        for k in range(n_chunks):
            p = k % nbuf
            idx_cps[k].wait()
            if k >= nbuf:
                out_cps[k - nbuf].wait()
            ib, ob = idx_bufs[p], out_bufs[p]

            @plsc.parallel_loop(0, _ROWS, unroll=4)
            def gather_body(r, ib=ib, ob=ob):
                for c in col_offs:
                    iv = ib[r, pl.ds(c, _LANES)]
                    ob[r, pl.ds(c, _LANES)] = plsc.load_gather(tab_v, [iv])

            out_cps[k] = pltpu.async_copy(
                ob, out_hbm.at[pl.ds(row0 + k * _ROWS, _ROWS), :], out_sems[p])
            if k + nbuf < n_chunks:
                idx_cps[k + nbuf] = pltpu.async_copy(
                    idx_hbm.at[pl.ds(row0 + (k + nbuf) * _ROWS, _ROWS), :],
                    idx_bufs[p], idx_sems[p])

        for k in range(n_chunks - nbuf, n_chunks):
            out_cps[k].wait()

    out = sc_gather(idx, table)
    return out
